# Initial kernel scaffold; baseline (speedup 1.0000x reference)
import jax
import jax.numpy as jnp
from jax.experimental import pallas as pl


def kernel(inp, h, Wk_in, Wq_in, Wv_in, W_ih, W_hh, b_ih, b_hh, Wq_c, Wk_c, Wv_c, Wo_c):
    raise NotImplementedError("write your pallas kernel here")


# speedup vs baseline: 2.4893x; 2.4893x over previous
import math

import jax
import jax.numpy as jnp
from jax.experimental import pallas as pl
from jax.experimental.pallas import tpu as pltpu

_T, _B = 32, 64
_NHID = 1024
_NB = 8
_BS = _NHID // _NB
_TOPK = 4
_DK = 64
_DC = 64
_NBB = _NB * _B
_G3 = 3 * _BS
_SQK = math.sqrt(_DK)
_SQC = math.sqrt(_DC)
_TT = 8
_KC = 256


def _dot(a, b):
    return jnp.dot(a, b, preferred_element_type=jnp.float32)


def _dot_kchunk(a, b):
    acc = jnp.zeros((a.shape[0], b.shape[1]), jnp.float32)
    for i in range(a.shape[1] // _KC):
        acc = acc + _dot(a[:, i * _KC:(i + 1) * _KC], b[i * _KC:(i + 1) * _KC, :])
    return acc


def _proj_kernel(x_ref, wk_ref, wv_ref, k_ref, v_ref):
    x = x_ref[...].reshape(_TT * _B, _NHID)
    k_ref[...] = _dot_kchunk(x, wk_ref[...])
    v_ref[...] = _dot_kchunk(x, wv_ref[...])


def _proj_call(inp, Wk_in, Wv_in):
    return pl.pallas_call(
        _proj_kernel,
        grid=(_T // _TT,),
        in_specs=[
            pl.BlockSpec((_TT, _B, _NHID), lambda i: (i, 0, 0)),
            pl.BlockSpec((_NHID, _DK), lambda i: (0, 0)),
            pl.BlockSpec((_NHID, _BS), lambda i: (0, 0)),
        ],
        out_specs=[
            pl.BlockSpec((_TT * _B, _DK), lambda i: (i, 0)),
            pl.BlockSpec((_TT * _B, _BS), lambda i: (i, 0)),
        ],
        out_shape=[
            jax.ShapeDtypeStruct((_T * _B, _DK), jnp.float32),
            jax.ShapeDtypeStruct((_T * _B, _BS), jnp.float32),
        ],
    )(inp, Wk_in, Wv_in)


def _scan_kernel(h0_ref, k_ref, v_ref, wq_ref, wih_ref, whh_ref, bih_ref,
                 bhh_ref, wqc_ref, wkc_ref, wvc_ref, woc_ref, cm_ref,
                 out_ref, h_scr):
    t = pl.program_id(0)

    @pl.when(t == 0)
    def _():
        h_scr[...] = h0_ref[...]

    h3 = h_scr[...]
    kk = k_ref[...]

    q3 = _dot(h3, wq_ref[...])
    sf = jax.lax.dot_general(q3, kk, (((1,), (1,)), ((), ())),
                             preferred_element_type=jnp.float32)
    row = jax.lax.broadcasted_iota(jnp.int32, (_NBB, _B), 0)
    col = jax.lax.broadcasted_iota(jnp.int32, (_NBB, _B), 1)
    s = jnp.sum(jnp.where(col == row % _B, sf, 0.0), axis=1, keepdims=True)
    s = s / _SQK
    m = jnp.maximum(s, 0.0)
    er = jnp.exp(s - m)
    en = jnp.exp(-m)
    a3 = er / (er + en)

    a = jnp.concatenate([a3[n * _B:(n + 1) * _B, :] for n in range(_NB)],
                        axis=1)
    cand = []
    for i in range(_NB):
        cnt = jnp.sum((a >= a[:, i:i + 1]).astype(jnp.float32), axis=1,
                      keepdims=True)
        cand.append(jnp.where(cnt >= _TOPK, a[:, i:i + 1], -1.0))
    thr = cand[0]
    for i in range(1, _NB):
        thr = jnp.maximum(thr, cand[i])
    mask3 = jnp.concatenate(
        [(a[:, n:n + 1] >= thr).astype(jnp.float32) for n in range(_NB)],
        axis=0)

    vv = v_ref[...]
    v3 = jnp.concatenate([vv] * _NB, axis=0)
    inp3 = a3 * v3
    gi = jnp.concatenate(
        [_dot(inp3[n * _B:(n + 1) * _B, :], wih_ref[n]) for n in range(_NB)],
        axis=0) + bih_ref[...]
    gh = jnp.concatenate(
        [_dot(h3[n * _B:(n + 1) * _B, :], whh_ref[n]) for n in range(_NB)],
        axis=0) + bhh_ref[...]
    r = jax.nn.sigmoid(gi[:, :_BS] + gh[:, :_BS])
    z = jax.nn.sigmoid(gi[:, _BS:2 * _BS] + gh[:, _BS:2 * _BS])
    nn = jnp.tanh(gi[:, 2 * _BS:] + r * gh[:, 2 * _BS:])
    hnew = (1.0 - z) * nn + z * h3

    qc = _dot(hnew, wqc_ref[...])
    kc = _dot(hnew, wkc_ref[...])
    vc = _dot(hnew, wvc_ref[...])
    s_full = jax.lax.dot_general(qc, kc, (((1,), (1,)), ((), ())),
                                 preferred_element_type=jnp.float32)
    s_full = jnp.where(cm_ref[...] > 0.0, s_full / _SQC, -jnp.inf)
    mm = jnp.max(s_full, axis=1, keepdims=True)
    e = jnp.exp(s_full - mm)
    ac = e / jnp.sum(e, axis=1, keepdims=True)
    comm = _dot(_dot(ac, vc), woc_ref[...])

    hnew = hnew + comm
    hfin = mask3 * hnew + (1.0 - mask3) * h3
    h_scr[...] = hfin
    for n in range(_NB):
        out_ref[0, :, n * _BS:(n + 1) * _BS] = hfin[n * _B:(n + 1) * _B, :]


def _scan_call(h0, k_all, v_all, Wq_in, W_ih, W_hh, bih3, bhh3,
               Wq_c, Wk_c, Wv_c, Wo_c, cm):
    return pl.pallas_call(
        _scan_kernel,
        grid=(_T,),
        in_specs=[
            pl.BlockSpec((_NBB, _BS), lambda t: (0, 0)),
            pl.BlockSpec((_B, _DK), lambda t: (t, 0)),
            pl.BlockSpec((_B, _BS), lambda t: (t, 0)),
            pl.BlockSpec((_BS, _DK), lambda t: (0, 0)),
            pl.BlockSpec((_NB, _BS, _G3), lambda t: (0, 0, 0)),
            pl.BlockSpec((_NB, _BS, _G3), lambda t: (0, 0, 0)),
            pl.BlockSpec((_NBB, _G3), lambda t: (0, 0)),
            pl.BlockSpec((_NBB, _G3), lambda t: (0, 0)),
            pl.BlockSpec((_BS, _DC), lambda t: (0, 0)),
            pl.BlockSpec((_BS, _DC), lambda t: (0, 0)),
            pl.BlockSpec((_BS, _DC), lambda t: (0, 0)),
            pl.BlockSpec((_DC, _BS), lambda t: (0, 0)),
            pl.BlockSpec((_NBB, _NBB), lambda t: (0, 0)),
        ],
        out_specs=pl.BlockSpec((1, _B, _NHID), lambda t: (t, 0, 0)),
        out_shape=jax.ShapeDtypeStruct((_T, _B, _NHID), jnp.float32),
        scratch_shapes=[pltpu.VMEM((_NBB, _BS), jnp.float32)],
    )(h0, k_all, v_all, Wq_in, W_ih, W_hh, bih3, bhh3,
      Wq_c, Wk_c, Wv_c, Wo_c, cm)


def kernel(inp, h, Wk_in, Wq_in, Wv_in, W_ih, W_hh, b_ih, b_hh,
           Wq_c, Wk_c, Wv_c, Wo_c):
    k_all, v_all = _proj_call(inp, Wk_in, Wv_in)

    h0 = h[0].reshape(_B, _NB, _BS).transpose(1, 0, 2).reshape(_NBB, _BS)
    bih3 = jnp.repeat(b_ih, _B, axis=0)
    bhh3 = jnp.repeat(b_hh, _B, axis=0)
    row = jax.lax.broadcasted_iota(jnp.int32, (_NBB, _NBB), 0)
    col = jax.lax.broadcasted_iota(jnp.int32, (_NBB, _NBB), 1)
    cm = ((row % _B) == (col % _B)).astype(jnp.float32)

    outs = _scan_call(h0, k_all, v_all, Wq_in, W_ih, W_hh, bih3, bhh3,
                      Wq_c, Wk_c, Wv_c, Wo_c, cm)
    return outs, outs[-1:]
